# baseline (speedup 1.0000x reference)
import numpy as np
import jax
import jax.numpy as jnp
from jax.experimental import pallas as pl
from jax.experimental.pallas import tpu as pltpu

F32 = jnp.float32


def _taps(ksize, p, q):
    return [((kh - p) // 2, (kw - q) // 2)
            for kh in range(p, ksize, 2) for kw in range(q, ksize, 2)]


_NT1 = {(p, q): len(_taps(7, p, q)) for p in range(2) for q in range(2)}
_NT2 = {(p, q): len(_taps(5, p, q)) for p in range(2) for q in range(2)}


def _k1_body(xb_ref, w_ee, w_eo, w_oe, w_oo, b_ref, out_ref):
    wrefs = {(0, 0): w_ee, (0, 1): w_eo, (1, 0): w_oe, (1, 1): w_oo}
    y = jnp.zeros((23, 112, 32), F32)
    for p in range(2):
        for q in range(2):
            slab = xb_ref[0, 0, p, q]
            z = jnp.dot(slab.reshape(26 * 116, 96), wrefs[(p, q)][...],
                        preferred_element_type=F32)
            z = z.reshape(26, 116, _NT1[(p, q)] * 32)
            for t, (sh, sw) in enumerate(_taps(7, p, q)):
                y = y + z[sh:sh + 23, sw:sw + 112, 32 * t:32 * t + 32]
    y = jnp.maximum(y + b_ref[...][None], 0.0)
    y4 = y.reshape(23, 56, 2, 32)
    ce, co = y4[:, :, 0], y4[:, :, 1]
    mw = jnp.maximum(jnp.maximum(ce[:, :55], co[:, :55]), ce[:, 1:56])
    pad = jnp.zeros((1, 55, 32), F32)
    m24 = jnp.concatenate([mw, pad], axis=0).reshape(12, 2, 55, 32)
    ev, od = m24[:, 0], m24[:, 1]
    out_ref[0] = jnp.maximum(jnp.maximum(ev[:11], od[:11]), ev[1:12])


def _k2_body(xb_ref, w_ee, w_eo, w_oe, w_oo, b_ref, out_ref):
    wrefs = {(0, 0): w_ee, (0, 1): w_eo, (1, 0): w_oe, (1, 1): w_oo}
    y = jnp.zeros((28, 28, 64), F32)
    for p in range(2):
        for q in range(2):
            slab = xb_ref[0, p, q]
            z = jnp.dot(slab.reshape(900, 32), wrefs[(p, q)][...],
                        preferred_element_type=F32)
            z = z.reshape(30, 30, _NT2[(p, q)] * 64)
            for t, (sh, sw) in enumerate(_taps(5, p, q)):
                y = y + z[sh:sh + 28, sw:sw + 28, 64 * t:64 * t + 64]
    y = jnp.maximum(y + b_ref[...][None], 0.0)
    s = y.reshape(4, 7, 28, 64).sum(axis=1).reshape(4, 4, 7, 64).sum(axis=2)
    out_ref[0] = (s * (1.0 / 49.0)).reshape(16, 64)


def _k3_body(f_ref, w1_ref, b1_ref, g_ref, bt_ref, w2_ref, b2_ref, t_ref,
             gates_ref, idx_ref, logits_ref):
    z = jnp.dot(f_ref[...], w1_ref[...], preferred_element_type=F32) + b1_ref[...]
    mu = jnp.mean(z, axis=1, keepdims=True)
    zc = z - mu
    var = jnp.mean(zc * zc, axis=1, keepdims=True)
    zn = zc / jnp.sqrt(var + 1e-5) * g_ref[...] + bt_ref[...]
    ge = 0.5 * zn * (1.0 + jax.lax.erf(zn * np.float32(1.0 / np.sqrt(2.0))))
    lg = jnp.dot(ge, w2_ref[...], preferred_element_type=F32) + b2_ref[...]
    t = jnp.clip(t_ref[0], 0.5, 5.0)
    lg = lg / t
    logits_ref[...] = lg
    iota = jax.lax.broadcasted_iota(jnp.int32, (16, 64), 1)
    big = jnp.int32(1 << 30)
    m1 = jnp.max(lg, axis=1, keepdims=True)
    i1 = jnp.min(jnp.where(lg == m1, iota, big), axis=1, keepdims=True)
    masked = jnp.where(iota == i1, -jnp.inf, lg)
    m2 = jnp.max(masked, axis=1, keepdims=True)
    i2 = jnp.min(jnp.where(masked == m2, iota, big), axis=1, keepdims=True)
    e2 = jnp.exp(m2 - m1)
    sm = 1.0 + e2
    g1 = 1.0 / sm
    g2 = e2 / sm
    ssum = g1 + g2
    g1 = g1 / (ssum + 1e-10)
    g2 = g2 / (ssum + 1e-10)
    gates_ref[...] = jnp.where(iota == i1, g1, 0.0) + jnp.where(iota == i2, g2, 0.0)
    idx_ref[...] = jnp.concatenate([i1, i2], axis=1)


def kernel(x, conv1_w, conv1_b, conv2_w, conv2_b, fc1_w, fc1_b,
           ln_g, ln_b, fc2_w, fc2_b, temperature):
    B = 16
    xp = jnp.pad(x, ((0, 0), (0, 0), (3, 5), (3, 5)))
    xph = xp.reshape(B, 96, 116, 2, 116, 2).transpose(0, 3, 5, 2, 4, 1)
    xb = jnp.stack([xph[:, :, :, 22 * m:22 * m + 26] for m in range(5)],
                   axis=1)
    w1s = {}
    for p in range(2):
        for q in range(2):
            cols = [conv1_w[:, :, kh, kw].T
                    for kh in range(p, 7, 2) for kw in range(q, 7, 2)]
            w1s[(p, q)] = jnp.concatenate(cols, axis=1)
    b1 = conv1_b.reshape(1, 32)

    mp = pl.pallas_call(
        _k1_body,
        grid=(B, 5),
        in_specs=[
            pl.BlockSpec((1, 1, 2, 2, 26, 116, 96), lambda b, m: (b, m, 0, 0, 0, 0, 0)),
            pl.BlockSpec((96, _NT1[(0, 0)] * 32), lambda b, m: (0, 0)),
            pl.BlockSpec((96, _NT1[(0, 1)] * 32), lambda b, m: (0, 0)),
            pl.BlockSpec((96, _NT1[(1, 0)] * 32), lambda b, m: (0, 0)),
            pl.BlockSpec((96, _NT1[(1, 1)] * 32), lambda b, m: (0, 0)),
            pl.BlockSpec((1, 32), lambda b, m: (0, 0)),
        ],
        out_specs=pl.BlockSpec((1, 11, 55, 32), lambda b, m: (b, m, 0, 0)),
        out_shape=jax.ShapeDtypeStruct((B, 55, 55, 32), F32),
        compiler_params=pltpu.CompilerParams(
            vmem_limit_bytes=100 * 1024 * 1024),
    )(xb, w1s[(0, 0)], w1s[(0, 1)], w1s[(1, 0)], w1s[(1, 1)], b1)

    mpp = jnp.pad(mp, ((0, 0), (2, 3), (2, 3), (0, 0)))
    mph = mpp.reshape(B, 30, 2, 30, 2, 32).transpose(0, 2, 4, 1, 3, 5)
    w2s = {}
    for p in range(2):
        for q in range(2):
            cols = [conv2_w[:, :, kh, kw].T
                    for kh in range(p, 5, 2) for kw in range(q, 5, 2)]
            w2s[(p, q)] = jnp.concatenate(cols, axis=1)
    b2 = conv2_b.reshape(1, 64)

    flat = pl.pallas_call(
        _k2_body,
        grid=(B,),
        in_specs=[
            pl.BlockSpec((1, 2, 2, 30, 30, 32), lambda b: (b, 0, 0, 0, 0, 0)),
            pl.BlockSpec((32, _NT2[(0, 0)] * 64), lambda b: (0, 0)),
            pl.BlockSpec((32, _NT2[(0, 1)] * 64), lambda b: (0, 0)),
            pl.BlockSpec((32, _NT2[(1, 0)] * 64), lambda b: (0, 0)),
            pl.BlockSpec((32, _NT2[(1, 1)] * 64), lambda b: (0, 0)),
            pl.BlockSpec((1, 64), lambda b: (0, 0)),
        ],
        out_specs=pl.BlockSpec((1, 16, 64), lambda b: (b, 0, 0)),
        out_shape=jax.ShapeDtypeStruct((B, 16, 64), F32),
    )(mph, w2s[(0, 0)], w2s[(0, 1)], w2s[(1, 0)], w2s[(1, 1)], b2)
    flat = flat.reshape(B, 1024)

    fc1_wp = fc1_w.reshape(128, 64, 4, 4).transpose(0, 2, 3, 1).reshape(128, 1024).T
    fc2_wt = fc2_w.T
    gates, idx, logits = pl.pallas_call(
        _k3_body,
        in_specs=[
            pl.BlockSpec((16, 1024), lambda: (0, 0)),
            pl.BlockSpec((1024, 128), lambda: (0, 0)),
            pl.BlockSpec((1, 128), lambda: (0, 0)),
            pl.BlockSpec((1, 128), lambda: (0, 0)),
            pl.BlockSpec((1, 128), lambda: (0, 0)),
            pl.BlockSpec((128, 64), lambda: (0, 0)),
            pl.BlockSpec((1, 64), lambda: (0, 0)),
            pl.BlockSpec(memory_space=pltpu.SMEM),
        ],
        out_specs=(
            pl.BlockSpec((16, 64), lambda: (0, 0)),
            pl.BlockSpec((16, 2), lambda: (0, 0)),
            pl.BlockSpec((16, 64), lambda: (0, 0)),
        ),
        out_shape=(
            jax.ShapeDtypeStruct((16, 64), F32),
            jax.ShapeDtypeStruct((16, 2), jnp.int32),
            jax.ShapeDtypeStruct((16, 64), F32),
        ),
    )(flat, fc1_wp, fc1_b.reshape(1, 128), ln_g.reshape(1, 128),
      ln_b.reshape(1, 128), fc2_wt, fc2_b.reshape(1, 64),
      temperature.reshape(1))
    return (gates, idx, logits)


# speedup vs baseline: 1.4171x; 1.4171x over previous
import numpy as np
import jax
import jax.numpy as jnp
from jax.experimental import pallas as pl
from jax.experimental.pallas import tpu as pltpu

F32 = jnp.float32


def _taps(ksize, p, q):
    return [((kh - p) // 2, (kw - q) // 2)
            for kh in range(p, ksize, 2) for kw in range(q, ksize, 2)]


_NT1 = {(p, q): len(_taps(7, p, q)) for p in range(2) for q in range(2)}
_BANDS = 8
_BR = 112 // _BANDS


def _conv_stages(w1_ref, b1_ref, w2e_ref, w2o_ref, b2_ref, out_ref,
                 tl_ref, yb_ref, mp_ref, s1_ref, z2s_ref):
    def _band(b, _):
        def _phase(f, acc):
            p = f // 2
            q = f % 2
            lhs = tl_ref[pl.Slice(28 * b + p, 18, 2),
                         pl.Slice(q, 116, 2), :]
            z = jnp.dot(lhs.reshape(18 * 116, 96),
                        w1_ref[pl.ds(f, 1)].reshape(96, 512),
                        preferred_element_type=F32)
            z3 = z.reshape(18, 116, 512)
            for t_ in range(16):
                sh, sw = t_ // 4, t_ % 4
                acc = acc + z3[sh:sh + 15, sw:sw + 112,
                               32 * t_:32 * t_ + 32]
            return acc

        acc = jax.lax.fori_loop(0, 4, _phase, jnp.zeros((15, 112, 32), F32))
        yb_ref[0:15] = jnp.maximum(acc + b1_ref[...], 0.0)
        ce = yb_ref[0:15, 0:111:2, :]
        co = yb_ref[0:15, 1:112:2, :]
        mw = jnp.maximum(jnp.maximum(ce[:, :55], co[:, :55]), ce[:, 1:56])
        m4 = mw[0:14].reshape(7, 2, 55, 32)
        e, o = m4[:, 0], m4[:, 1]
        e2 = jnp.concatenate([e[1:7], mw[14:15]], axis=0)
        mp_ref[pl.ds(2 + 7 * b, 7), 2:57, :] = \
            jnp.maximum(jnp.maximum(e, o), e2)
        return 0

    jax.lax.fori_loop(0, _BANDS, _band, 0)
    mp_ref[0:2, :, :] = jnp.zeros((2, 60, 32), F32)
    mp_ref[57:60, :, :] = jnp.zeros((3, 60, 32), F32)
    mp_ref[:, 0:2, :] = jnp.zeros((60, 2, 32), F32)
    mp_ref[:, 57:60, :] = jnp.zeros((60, 3, 32), F32)
    lhs_q = {0: mp_ref[:, 0:59:2, :].reshape(1800, 32),
             1: mp_ref[:, 1:60:2, :].reshape(1800, 32)}
    w2refs = {0: w2e_ref, 1: w2o_ref}
    y2 = jnp.zeros((28, 28, 64), F32)
    for q in range(2):
        kws = list(range(q, 5, 2))
        n = 64 * len(kws)

        def _kh(kh, y2_, q=q, kws=kws, n=n):
            z2 = jnp.dot(lhs_q[q], w2refs[q][kh],
                         preferred_element_type=F32)
            z3 = z2.reshape(60, 30, n)
            for t_ in range(len(kws)):
                z2s_ref[t_] = z3[:, :, 64 * t_:64 * t_ + 64]
            for t_, kw in enumerate(kws):
                sw = (kw - q) // 2
                v = z2s_ref[pl.ds(t_, 1), pl.Slice(kh, 28, 2), :, :]
                y2_ = y2_ + v.reshape(28, 30, 64)[:, sw:sw + 28, :]
            return y2_

        y2 = jax.lax.fori_loop(0, 5, _kh, y2)
    y2 = jnp.maximum(y2 + b2_ref[...], 0.0)
    s1_ref[...] = y2.reshape(4, 7, 28, 64).sum(axis=1)
    s2 = jnp.zeros((4, 4, 64), F32)
    for d in range(7):
        s2 = s2 + s1_ref[:, d:d + 22:7, :]
    out_ref[0] = (s2 * (1.0 / 49.0)).reshape(16, 64)


def _k1_body(x_ref, w1_ref, b1_ref, w2e_ref, w2o_ref, b2_ref,
             out_ref, tl_ref, yb_ref, mp_ref, s1_ref, z2s_ref):
    rc = pl.program_id(1)

    @pl.when(rc == 0)
    def _zero():
        tl_ref[0:3, :, :] = jnp.zeros((3, 232, 96), F32)
        tl_ref[227:232, :, :] = jnp.zeros((5, 232, 96), F32)
        tl_ref[:, 0:3, :] = jnp.zeros((232, 3, 96), F32)
        tl_ref[:, 227:232, :] = jnp.zeros((232, 5, 96), F32)

    for g in range(2):
        chunk = x_ref[0, :, :, 112 * g:112 * g + 112]
        t = jnp.transpose(chunk, (1, 2, 0))
        tl_ref[pl.ds(3 + 32 * rc, 32), 3 + 112 * g:115 + 112 * g, :] = t

    @pl.when(rc == 6)
    def _compute():
        _conv_stages(w1_ref, b1_ref, w2e_ref, w2o_ref, b2_ref, out_ref,
                     tl_ref, yb_ref, mp_ref, s1_ref, z2s_ref)


def _k3_body(f_ref, w1_ref, b1_ref, g_ref, bt_ref, w2_ref, b2_ref, t_ref,
             gates_ref, idx_ref, logits_ref):
    z = jnp.dot(f_ref[...], w1_ref[...], preferred_element_type=F32) + b1_ref[...]
    mu = jnp.mean(z, axis=1, keepdims=True)
    zc = z - mu
    var = jnp.mean(zc * zc, axis=1, keepdims=True)
    zn = zc / jnp.sqrt(var + 1e-5) * g_ref[...] + bt_ref[...]
    ge = 0.5 * zn * (1.0 + jax.lax.erf(zn * np.float32(1.0 / np.sqrt(2.0))))
    lg = jnp.dot(ge, w2_ref[...], preferred_element_type=F32) + b2_ref[...]
    t = jnp.clip(t_ref[0], 0.5, 5.0)
    lg = lg / t
    logits_ref[...] = lg
    iota = jax.lax.broadcasted_iota(jnp.int32, (16, 64), 1)
    big = jnp.int32(1 << 30)
    m1 = jnp.max(lg, axis=1, keepdims=True)
    i1 = jnp.min(jnp.where(lg == m1, iota, big), axis=1, keepdims=True)
    masked = jnp.where(iota == i1, -jnp.inf, lg)
    m2 = jnp.max(masked, axis=1, keepdims=True)
    i2 = jnp.min(jnp.where(masked == m2, iota, big), axis=1, keepdims=True)
    e2 = jnp.exp(m2 - m1)
    sm = 1.0 + e2
    g1 = 1.0 / sm
    g2 = e2 / sm
    ssum = g1 + g2
    g1 = g1 / (ssum + 1e-10)
    g2 = g2 / (ssum + 1e-10)
    gates_ref[...] = jnp.where(iota == i1, g1, 0.0) + jnp.where(iota == i2, g2, 0.0)
    idx_ref[...] = jnp.concatenate([i1, i2], axis=1)


def kernel(x, conv1_w, conv1_b, conv2_w, conv2_b, fc1_w, fc1_b,
           ln_g, ln_b, fc2_w, fc2_b, temperature):
    B = 16
    zblk = jnp.zeros((96, 32), x.dtype)
    phs = []
    for p in range(2):
        for q in range(2):
            cols = []
            for sh in range(4):
                for sw in range(4):
                    kh, kw = p + 2 * sh, q + 2 * sw
                    cols.append(conv1_w[:, :, kh, kw].T
                                if (kh < 7 and kw < 7) else zblk)
            phs.append(jnp.concatenate(cols, axis=1))
    w1all = jnp.stack(phs, axis=0)
    w2q = {}
    for q in range(2):
        w2q[q] = jnp.stack(
            [jnp.concatenate([conv2_w[:, :, kh, kw].T
                              for kw in range(q, 5, 2)], axis=1)
             for kh in range(5)], axis=0)
    b1 = conv1_b.reshape(1, 1, 32)
    b2 = conv2_b.reshape(1, 1, 64)

    flat = pl.pallas_call(
        _k1_body,
        grid=(B, 7),
        in_specs=[
            pl.BlockSpec((1, 96, 32, 224), lambda b, rc: (b, 0, rc, 0)),
            pl.BlockSpec((4, 96, 512), lambda b, rc: (0, 0, 0)),
            pl.BlockSpec((1, 1, 32), lambda b, rc: (0, 0, 0)),
            pl.BlockSpec((5, 32, 192), lambda b, rc: (0, 0, 0)),
            pl.BlockSpec((5, 32, 128), lambda b, rc: (0, 0, 0)),
            pl.BlockSpec((1, 1, 64), lambda b, rc: (0, 0, 0)),
        ],
        out_specs=pl.BlockSpec((1, 16, 64), lambda b, rc: (b, 0, 0)),
        out_shape=jax.ShapeDtypeStruct((B, 16, 64), F32),
        scratch_shapes=[
            pltpu.VMEM((232, 232, 96), F32),
            pltpu.VMEM((16, 112, 32), F32),
            pltpu.VMEM((60, 60, 32), F32),
            pltpu.VMEM((4, 28, 64), F32),
            pltpu.VMEM((3, 60, 30, 64), F32),
        ],
        compiler_params=pltpu.CompilerParams(
            vmem_limit_bytes=66 * 1024 * 1024),
    )(x, w1all, b1, w2q[0], w2q[1], b2)
    flat = flat.reshape(B, 1024)

    fc1_wp = fc1_w.reshape(128, 64, 4, 4).transpose(0, 2, 3, 1).reshape(128, 1024).T
    fc2_wt = fc2_w.T
    gates, idx, logits = pl.pallas_call(
        _k3_body,
        in_specs=[
            pl.BlockSpec((16, 1024), lambda: (0, 0)),
            pl.BlockSpec((1024, 128), lambda: (0, 0)),
            pl.BlockSpec((1, 128), lambda: (0, 0)),
            pl.BlockSpec((1, 128), lambda: (0, 0)),
            pl.BlockSpec((1, 128), lambda: (0, 0)),
            pl.BlockSpec((128, 64), lambda: (0, 0)),
            pl.BlockSpec((1, 64), lambda: (0, 0)),
            pl.BlockSpec(memory_space=pltpu.SMEM),
        ],
        out_specs=(
            pl.BlockSpec((16, 64), lambda: (0, 0)),
            pl.BlockSpec((16, 2), lambda: (0, 0)),
            pl.BlockSpec((16, 64), lambda: (0, 0)),
        ),
        out_shape=(
            jax.ShapeDtypeStruct((16, 64), F32),
            jax.ShapeDtypeStruct((16, 2), jnp.int32),
            jax.ShapeDtypeStruct((16, 64), F32),
        ),
    )(flat, fc1_wp, fc1_b.reshape(1, 128), ln_g.reshape(1, 128),
      ln_b.reshape(1, 128), fc2_wt, fc2_b.reshape(1, 64),
      temperature.reshape(1))
    return (gates, idx, logits)
